# baseline (speedup 1.0000x reference)
import functools

import jax
import jax.numpy as jnp
from jax import lax
from jax.experimental import pallas as pl
from jax.experimental.pallas import tpu as pltpu
from jax.experimental.pallas import tpu_sc as plsc

E = 1000
L = 250
D = 256
B = 4096
T = 50
NC = 2
NS = 16
NW = NC * NS
BPW = B // NW
RPW = BPW * T
CHUNK = 128
NCHUNK = RPW // CHUNK


def _indices_kernel(ep_ref, st_ref, out_ref):
    t = lax.broadcasted_iota(jnp.int32, (B, T), 1)
    out_ref[...] = ep_ref[...] * L + st_ref[...] + t


def _flat_indices(episodes, start):
    return pl.pallas_call(
        _indices_kernel,
        out_shape=jax.ShapeDtypeStruct((B, T), jnp.int32),
    )(episodes.reshape(B, 1), start.reshape(B, 1))


def _sc_gather(obs_flat, idx3):
    mesh = plsc.VectorSubcoreMesh(core_axis_name="c", subcore_axis_name="s")

    @functools.partial(
        pl.kernel,
        mesh=mesh,
        out_type=jax.ShapeDtypeStruct((NW * NCHUNK, CHUNK, D), jnp.float32),
        scratch_types=[
            pltpu.VMEM((NCHUNK, CHUNK), jnp.int32),
            pltpu.VMEM((CHUNK, D), jnp.float32),
            pltpu.VMEM((CHUNK, D), jnp.float32),
            pltpu.SemaphoreType.DMA,
            pltpu.SemaphoreType.DMA,
            pltpu.SemaphoreType.DMA,
            pltpu.SemaphoreType.DMA,
        ],
    )
    def k(obs_hbm, idx_hbm, obs_out, idx_v, buf0, buf1, g0, g1, s0, s1):
        w = lax.axis_index("s") * NC + lax.axis_index("c")
        pltpu.sync_copy(idx_hbm.at[w], idx_v)

        bufs = (buf0, buf1)
        gsems = (g0, g1)
        ssems = (s0, s1)

        def gcopy(c):
            p = c % 2
            return pltpu.make_async_copy(
                obs_hbm.at[idx_v.at[c]], bufs[p], gsems[p])

        def scopy(c):
            p = c % 2
            return pltpu.make_async_copy(
                bufs[p], obs_out.at[w * NCHUNK + c], ssems[p])

        gcopy(0).start()
        for c in range(NCHUNK):
            if c + 1 < NCHUNK:
                if c >= 1:
                    scopy(c - 1).wait()
                gcopy(c + 1).start()
            gcopy(c).wait()
            scopy(c).start()
        scopy(NCHUNK - 2).wait()
        scopy(NCHUNK - 1).wait()

    return k(obs_flat, idx3)


def kernel(obs, episodes, start, trajectory_len):
    del trajectory_len
    idx = _flat_indices(episodes, start)
    obs_flat = obs.reshape(E * L, D)
    rows = _sc_gather(obs_flat, idx.reshape(NW, NCHUNK, CHUNK))
    return idx, rows.reshape(B, T, D)
